# Initial kernel scaffold; baseline (speedup 1.0000x reference)
import jax
import jax.numpy as jnp
from jax.experimental import pallas as pl


def kernel(x, edge_index, W1, a_src1, a_dst1, b1, W2, a_src2, a_dst2, b2):
    raise NotImplementedError("write your pallas kernel here")


# speedup vs baseline: 69.1770x; 69.1770x over previous
import functools

import jax
import jax.numpy as jnp
from jax import lax
from jax.experimental import pallas as pl
from jax.experimental.pallas import tpu as pltpu
from jax.experimental.pallas import tpu_sc as plsc

N = 10000
E = 320000
NFEAT = 128
NHID = 8
NHEADS = 8
NCLASS = 40

NTILE = 32
EPT = E // NTILE
C = 80
NCH = EPT // C
ROWS1 = 72
ROWS2 = 48
NP = 10240
RPT = NP // 16

_BN = 1000


def _tc1_body(x_ref, w1_ref, as_ref, ad_ref, t1_ref, a1_ref):
    h = jnp.dot(x_ref[...], w1_ref[...], preferred_element_type=jnp.float32)
    als = jnp.dot(h, as_ref[...], preferred_element_type=jnp.float32)
    ald = jnp.dot(h, ad_ref[...], preferred_element_type=jnp.float32)
    t1_ref[...] = jnp.concatenate([h, als], axis=1)
    a1_ref[...] = ald


def _tc2_body(acc_ref, b1_ref, w2_ref, as2_ref, ad2_ref, e8_ref, t2_ref, a2_ref):
    a0 = acc_ref[0]
    a1 = acc_ref[1]
    num = a0[:, :64] + a1[:, :64]
    den = a0[:, 64:] + a1[:, 64:]
    r = 1.0 / (den + 1e-16)
    rexp = jnp.dot(r, e8_ref[...], preferred_element_type=jnp.float32)
    hid = num * rexp + b1_ref[...]
    hid = jnp.where(hid > 0, hid, jnp.exp(hid) - 1.0)
    h2 = jnp.dot(hid, w2_ref[...], preferred_element_type=jnp.float32)
    als2 = jnp.sum(h2 * as2_ref[...], axis=1, keepdims=True)
    ald2 = jnp.sum(h2 * ad2_ref[...], axis=1, keepdims=True)
    pad = jnp.zeros((h2.shape[0], 7), jnp.float32)
    t2_ref[...] = jnp.concatenate([h2, als2, pad], axis=1)
    a2_ref[...] = jnp.broadcast_to(ald2, (h2.shape[0], 8))


def _tc3_body(acc_ref, b2_ref, out_ref):
    a0 = acc_ref[0]
    a1 = acc_ref[1]
    num = a0[:, :40] + a1[:, :40]
    den = a0[:, 40:41] + a1[:, 40:41]
    o = num / (den + 1e-16) + b2_ref[...]
    m = jnp.max(o, axis=1, keepdims=True)
    s = jnp.sum(jnp.exp(o - m), axis=1, keepdims=True)
    out_ref[...] = o - m - jnp.log(s)


def _iota16():
    return lax.broadcasted_iota(jnp.int32, (16,), 0)


def _perm16(x, idx):
    return lax.gather(
        x, idx[:, None],
        lax.GatherDimensionNumbers(offset_dims=(), collapsed_slice_dims=(0,),
                                   start_index_map=(0,)),
        (1,), mode=lax.GatherScatterMode.PROMISE_IN_BOUNDS)


_mesh = plsc.VectorSubcoreMesh(core_axis_name="c", subcore_axis_name="s",
                               num_cores=2, num_subcores=16)


@functools.partial(
    pl.kernel,
    out_type=jax.ShapeDtypeStruct((2, NP, ROWS1), jnp.float32),
    mesh=_mesh,
    compiler_params=pltpu.CompilerParams(use_tc_tiling_on_sc=False,
                                         needs_layout_passes=False),
    scratch_types=[
        pltpu.VMEM_SHARED((NP, ROWS1), jnp.float32),
        pltpu.VMEM((EPT,), jnp.int32),
        pltpu.VMEM((EPT,), jnp.int32),
        pltpu.VMEM((C,), jnp.int32),
        pltpu.VMEM((C, ROWS1), jnp.float32),
        pltpu.VMEM((C, 8), jnp.float32),
        pltpu.VMEM((C, ROWS1), jnp.float32),
        pltpu.SemaphoreType.DMA,
    ],
)
def _sc1(src_ref, dst_ref, t1_ref, a1_ref, z_ref, out_ref,
         acc, src_v, dst_v, dstc_v, rows_v, ad_v, out_v, sem):
    cid = lax.axis_index("c")
    sid = lax.axis_index("s")
    wid = cid * 16 + sid

    pltpu.sync_copy(z_ref, acc.at[pl.ds(sid * RPT, RPT)])

    pltpu.sync_copy(src_ref.at[pl.ds(wid * EPT, EPT)], src_v)
    pltpu.sync_copy(dst_ref.at[pl.ds(wid * EPT, EPT)], dst_v)
    plsc.subcore_barrier()

    def chunk(k, _):
        lanes = _iota16()
        half = lanes // 8
        col8 = lanes & 7
        off = k * C
        pltpu.async_copy(t1_ref.at[src_v.at[pl.ds(off, C)]], rows_v, sem).wait()
        pltpu.async_copy(a1_ref.at[dst_v.at[pl.ds(off, C)]], ad_v, sem).wait()
        for j in range(C // 16):
            dstc_v[pl.ds(j * 16, 16)] = dst_v[pl.ds(off + j * 16, 16)]
        for p in range(C // 2):
            idx_r = 2 * p + half
            als = plsc.load_gather(rows_v, [idx_r, 64 + col8])
            ald = plsc.load_gather(ad_v, [idx_r, col8])
            t = als + ald
            w = jnp.exp(jnp.maximum(t, 0.2 * t))
            plsc.store_scatter(out_v, [idx_r, 64 + col8], w)
            for e in range(2):
                row = 2 * p + e
                for kk in range(4):
                    hv = rows_v[row, pl.ds(kk * 16, 16)]
                    wb = _perm16(w, e * 8 + 2 * kk + half)
                    out_v[row, pl.ds(kk * 16, 16)] = hv * wb
        pltpu.sync_copy(out_v, acc.at[dstc_v], add=True)
        return 0

    lax.fori_loop(0, NCH, chunk, 0)
    plsc.subcore_barrier()
    base = sid * RPT
    pltpu.sync_copy(acc.at[pl.ds(base, RPT)],
                    out_ref.at[cid, pl.ds(base, RPT)])


@functools.partial(
    pl.kernel,
    out_type=jax.ShapeDtypeStruct((2, NP, ROWS2), jnp.float32),
    mesh=_mesh,
    compiler_params=pltpu.CompilerParams(use_tc_tiling_on_sc=False,
                                         needs_layout_passes=False),
    scratch_types=[
        pltpu.VMEM_SHARED((NP, ROWS2), jnp.float32),
        pltpu.VMEM((EPT,), jnp.int32),
        pltpu.VMEM((EPT,), jnp.int32),
        pltpu.VMEM((C,), jnp.int32),
        pltpu.VMEM((C, ROWS2), jnp.float32),
        pltpu.VMEM((C, 8), jnp.float32),
        pltpu.VMEM((C, ROWS2), jnp.float32),
        pltpu.SemaphoreType.DMA,
    ],
)
def _sc2(src_ref, dst_ref, t2_ref, a2_ref, z_ref, out_ref,
         acc, src_v, dst_v, dstc_v, rows_v, ad_v, out_v, sem):
    cid = lax.axis_index("c")
    sid = lax.axis_index("s")
    wid = cid * 16 + sid

    pltpu.sync_copy(z_ref, acc.at[pl.ds(sid * RPT, RPT)])
    pltpu.sync_copy(src_ref.at[pl.ds(wid * EPT, EPT)], src_v)
    pltpu.sync_copy(dst_ref.at[pl.ds(wid * EPT, EPT)], dst_v)
    plsc.subcore_barrier()

    def chunk(k, _):
        lanes = _iota16()
        c40 = jnp.full((16,), 40, jnp.int32)
        c0 = jnp.zeros((16,), jnp.int32)
        off = k * C
        pltpu.async_copy(t2_ref.at[src_v.at[pl.ds(off, C)]], rows_v, sem).wait()
        pltpu.async_copy(a2_ref.at[dst_v.at[pl.ds(off, C)]], ad_v, sem).wait()
        for j in range(C // 16):
            dstc_v[pl.ds(j * 16, 16)] = dst_v[pl.ds(off + j * 16, 16)]
        for g in range(C // 16):
            idx_r = g * 16 + lanes
            als = plsc.load_gather(rows_v, [idx_r, c40])
            ald = plsc.load_gather(ad_v, [idx_r, c0])
            t = als + ald
            w = jnp.exp(jnp.maximum(t, 0.2 * t))
            for e in range(16):
                row = g * 16 + e
                wb = _perm16(w, jnp.full((16,), e, jnp.int32))
                for kk in range(3):
                    hv = rows_v[row, pl.ds(kk * 16, 16)]
                    out_v[row, pl.ds(kk * 16, 16)] = hv * wb
            plsc.store_scatter(out_v, [idx_r, c40], w)
        pltpu.sync_copy(out_v, acc.at[dstc_v], add=True)
        return 0

    lax.fori_loop(0, NCH, chunk, 0)
    plsc.subcore_barrier()
    base = sid * RPT
    pltpu.sync_copy(acc.at[pl.ds(base, RPT)],
                    out_ref.at[cid, pl.ds(base, RPT)])


def kernel(x, edge_index, W1, a_src1, a_dst1, b1, W2, a_src2, a_dst2, b2):
    f32 = jnp.float32
    sel = (jnp.arange(64)[:, None] // 8) == jnp.arange(8)[None, :]
    As = jnp.where(sel, a_src1.reshape(64)[:, None], 0.0).astype(f32)
    Ad = jnp.where(sel, a_dst1.reshape(64)[:, None], 0.0).astype(f32)
    E8 = sel.astype(f32).T

    T1, A1 = pl.pallas_call(
        _tc1_body,
        grid=(N // _BN,),
        in_specs=[
            pl.BlockSpec((_BN, NFEAT), lambda i: (i, 0)),
            pl.BlockSpec((NFEAT, 64), lambda i: (0, 0)),
            pl.BlockSpec((64, 8), lambda i: (0, 0)),
            pl.BlockSpec((64, 8), lambda i: (0, 0)),
        ],
        out_specs=[
            pl.BlockSpec((_BN, ROWS1), lambda i: (i, 0)),
            pl.BlockSpec((_BN, 8), lambda i: (i, 0)),
        ],
        out_shape=[
            jax.ShapeDtypeStruct((N, ROWS1), f32),
            jax.ShapeDtypeStruct((N, 8), f32),
        ],
    )(x, W1, As, Ad)

    src = edge_index[0]
    dst = edge_index[1]
    z1 = jnp.zeros((RPT, ROWS1), f32)
    acc1 = _sc1(src, dst, T1, A1, z1)

    T2, A2 = pl.pallas_call(
        _tc2_body,
        grid=(N // _BN,),
        in_specs=[
            pl.BlockSpec((2, _BN, ROWS1), lambda i: (0, i, 0)),
            pl.BlockSpec((1, 64), lambda i: (0, 0)),
            pl.BlockSpec((64, NCLASS), lambda i: (0, 0)),
            pl.BlockSpec((1, NCLASS), lambda i: (0, 0)),
            pl.BlockSpec((1, NCLASS), lambda i: (0, 0)),
            pl.BlockSpec((8, 64), lambda i: (0, 0)),
        ],
        out_specs=[
            pl.BlockSpec((_BN, ROWS2), lambda i: (i, 0)),
            pl.BlockSpec((_BN, 8), lambda i: (i, 0)),
        ],
        out_shape=[
            jax.ShapeDtypeStruct((N, ROWS2), f32),
            jax.ShapeDtypeStruct((N, 8), f32),
        ],
    )(acc1, b1.reshape(1, 64), W2, a_src2, a_dst2, E8)

    z2 = jnp.zeros((RPT, ROWS2), f32)
    acc2 = _sc2(src, dst, T2, A2, z2)

    out = pl.pallas_call(
        _tc3_body,
        grid=(N // _BN,),
        in_specs=[
            pl.BlockSpec((2, _BN, ROWS2), lambda i: (0, i, 0)),
            pl.BlockSpec((1, NCLASS), lambda i: (0, 0)),
        ],
        out_specs=pl.BlockSpec((_BN, NCLASS), lambda i: (i, 0)),
        out_shape=jax.ShapeDtypeStruct((N, NCLASS), f32),
    )(acc2, b2.reshape(1, NCLASS))

    return out


# speedup vs baseline: 120.8475x; 1.7469x over previous
import functools

import jax
import jax.numpy as jnp
from jax import lax
from jax.experimental import pallas as pl
from jax.experimental.pallas import tpu as pltpu
from jax.experimental.pallas import tpu_sc as plsc

N = 10000
E = 320000
NFEAT = 128
NHID = 8
NHEADS = 8
NCLASS = 40

NTILE = 32
EPT = E // NTILE
C = 80
NCH = EPT // C
ROWS1 = 72
ROWS2 = 48
NP = 10240
RPT = NP // 16

_BN = 1000


def _tc1_body(x_ref, w1_ref, as_ref, ad_ref, t1_ref, a1_ref):
    h = jnp.dot(x_ref[...], w1_ref[...], preferred_element_type=jnp.float32)
    als = jnp.dot(h, as_ref[...], preferred_element_type=jnp.float32)
    ald = jnp.dot(h, ad_ref[...], preferred_element_type=jnp.float32)
    t1_ref[...] = jnp.concatenate([h, als], axis=1)
    a1_ref[...] = ald


def _tc2_body(acc_ref, b1_ref, w2_ref, as2_ref, ad2_ref, e8_ref, t2_ref, a2_ref):
    a0 = acc_ref[0]
    a1 = acc_ref[1]
    num = a0[:, :64] + a1[:, :64]
    den = a0[:, 64:] + a1[:, 64:]
    r = 1.0 / (den + 1e-16)
    rexp = jnp.dot(r, e8_ref[...], preferred_element_type=jnp.float32)
    hid = num * rexp + b1_ref[...]
    hid = jnp.where(hid > 0, hid, jnp.exp(hid) - 1.0)
    h2 = jnp.dot(hid, w2_ref[...], preferred_element_type=jnp.float32)
    als2 = jnp.sum(h2 * as2_ref[...], axis=1, keepdims=True)
    ald2 = jnp.sum(h2 * ad2_ref[...], axis=1, keepdims=True)
    pad = jnp.zeros((h2.shape[0], 7), jnp.float32)
    t2_ref[...] = jnp.concatenate([h2, als2, pad], axis=1)
    a2_ref[...] = jnp.broadcast_to(ald2, (h2.shape[0], 8))


def _tc3_body(acc_ref, b2_ref, out_ref):
    a0 = acc_ref[0]
    a1 = acc_ref[1]
    num = a0[:, :40] + a1[:, :40]
    den = a0[:, 40:41] + a1[:, 40:41]
    o = num / (den + 1e-16) + b2_ref[...]
    m = jnp.max(o, axis=1, keepdims=True)
    s = jnp.sum(jnp.exp(o - m), axis=1, keepdims=True)
    out_ref[...] = o - m - jnp.log(s)


def _iota16():
    return lax.broadcasted_iota(jnp.int32, (16,), 0)


def _perm16(x, idx):
    return lax.gather(
        x, idx[:, None],
        lax.GatherDimensionNumbers(offset_dims=(), collapsed_slice_dims=(0,),
                                   start_index_map=(0,)),
        (1,), mode=lax.GatherScatterMode.PROMISE_IN_BOUNDS)


_mesh = plsc.VectorSubcoreMesh(core_axis_name="c", subcore_axis_name="s",
                               num_cores=2, num_subcores=16)


@functools.partial(
    pl.kernel,
    out_type=jax.ShapeDtypeStruct((2, NP, ROWS1), jnp.float32),
    mesh=_mesh,
    compiler_params=pltpu.CompilerParams(use_tc_tiling_on_sc=False,
                                         needs_layout_passes=False),
    scratch_types=[
        pltpu.VMEM_SHARED((NP, ROWS1), jnp.float32),
        pltpu.VMEM((EPT,), jnp.int32),
        pltpu.VMEM((EPT,), jnp.int32),
        pltpu.VMEM((C,), jnp.int32),
        pltpu.VMEM((C,), jnp.int32),
        pltpu.VMEM((C, ROWS1), jnp.float32),
        pltpu.VMEM((C, ROWS1), jnp.float32),
        pltpu.VMEM((C, 8), jnp.float32),
        pltpu.VMEM((C, 8), jnp.float32),
        pltpu.VMEM((C, ROWS1), jnp.float32),
        pltpu.VMEM((C, ROWS1), jnp.float32),
        pltpu.SemaphoreType.DMA,
        pltpu.SemaphoreType.DMA,
        pltpu.SemaphoreType.DMA,
        pltpu.SemaphoreType.DMA,
    ],
)
def _sc1(src_ref, dst_ref, t1_ref, a1_ref, z_ref, out_ref,
         acc, src_v, dst_v, dstc0, dstc1, rows0, rows1, ad0, ad1,
         out0, out1, gs0, gs1, ss0, ss1):
    cid = lax.axis_index("c")
    sid = lax.axis_index("s")
    wid = cid * 16 + sid
    dstc = (dstc0, dstc1)
    rows = (rows0, rows1)
    ad = (ad0, ad1)
    out = (out0, out1)
    gs = (gs0, gs1)
    ss = (ss0, ss1)

    pltpu.sync_copy(z_ref, acc.at[pl.ds(sid * RPT, RPT)])

    pltpu.sync_copy(src_ref.at[pl.ds(wid * EPT, EPT)], src_v)
    pltpu.sync_copy(dst_ref.at[pl.ds(wid * EPT, EPT)], dst_v)
    plsc.subcore_barrier()

    def g_start(cc, b):
        off = cc * C
        pltpu.async_copy(t1_ref.at[src_v.at[pl.ds(off, C)]], rows[b], gs[b])
        pltpu.async_copy(a1_ref.at[dst_v.at[pl.ds(off, C)]], ad[b], gs[b])

    def g_wait(b):
        pltpu.make_async_copy(t1_ref.at[pl.ds(0, C)], rows[b], gs[b]).wait()
        pltpu.make_async_copy(a1_ref.at[pl.ds(0, C)], ad[b], gs[b]).wait()

    def s_start(b):
        pltpu.async_copy(out[b], acc.at[dstc[b]], ss[b], add=True)

    def s_wait(b):
        pltpu.make_async_copy(out[b], acc.at[dstc[b]], ss[b]).wait()

    def compute(cc, b):
        lanes = _iota16()
        half = lanes // 8
        col8 = lanes & 7
        off = cc * C
        for j in range(C // 16):
            dstc[b][pl.ds(j * 16, 16)] = dst_v[pl.ds(off + j * 16, 16)]
        for p in range(C // 2):
            idx_r = 2 * p + half
            als = plsc.load_gather(rows[b], [idx_r, 64 + col8])
            ald = plsc.load_gather(ad[b], [idx_r, col8])
            t = als + ald
            w = jnp.exp(jnp.maximum(t, 0.2 * t))
            plsc.store_scatter(out[b], [idx_r, 64 + col8], w)
            for e in range(2):
                row = 2 * p + e
                for kk in range(4):
                    hv = rows[b][row, pl.ds(kk * 16, 16)]
                    wb = _perm16(w, e * 8 + 2 * kk + half)
                    out[b][row, pl.ds(kk * 16, 16)] = hv * wb

    g_start(0, 0)
    g_start(1, 1)

    def step(k2, _):
        for b in range(2):
            cc = 2 * k2 + b

            @pl.when(cc < NCH)
            def _():
                g_wait(b)

                @pl.when(cc >= 2)
                def _():
                    s_wait(b)

                compute(cc, b)
                s_start(b)

                @pl.when(cc + 2 < NCH)
                def _():
                    g_start(cc + 2, b)

        return 0

    lax.fori_loop(0, (NCH + 1) // 2, step, 0)
    s_wait(1)
    s_wait(0)
    plsc.subcore_barrier()
    base = sid * RPT
    pltpu.sync_copy(acc.at[pl.ds(base, RPT)],
                    out_ref.at[cid, pl.ds(base, RPT)])


@functools.partial(
    pl.kernel,
    out_type=jax.ShapeDtypeStruct((2, NP, ROWS2), jnp.float32),
    mesh=_mesh,
    compiler_params=pltpu.CompilerParams(use_tc_tiling_on_sc=False,
                                         needs_layout_passes=False),
    scratch_types=[
        pltpu.VMEM_SHARED((NP, ROWS2), jnp.float32),
        pltpu.VMEM((EPT,), jnp.int32),
        pltpu.VMEM((EPT,), jnp.int32),
        pltpu.VMEM((C,), jnp.int32),
        pltpu.VMEM((C,), jnp.int32),
        pltpu.VMEM((C, ROWS2), jnp.float32),
        pltpu.VMEM((C, ROWS2), jnp.float32),
        pltpu.VMEM((C, 8), jnp.float32),
        pltpu.VMEM((C, 8), jnp.float32),
        pltpu.VMEM((C, ROWS2), jnp.float32),
        pltpu.VMEM((C, ROWS2), jnp.float32),
        pltpu.SemaphoreType.DMA,
        pltpu.SemaphoreType.DMA,
        pltpu.SemaphoreType.DMA,
        pltpu.SemaphoreType.DMA,
    ],
)
def _sc2(src_ref, dst_ref, t2_ref, a2_ref, z_ref, out_ref,
         acc, src_v, dst_v, dstc0, dstc1, rows0, rows1, ad0, ad1,
         out0, out1, gs0, gs1, ss0, ss1):
    cid = lax.axis_index("c")
    sid = lax.axis_index("s")
    wid = cid * 16 + sid
    dstc = (dstc0, dstc1)
    rows = (rows0, rows1)
    ad = (ad0, ad1)
    out = (out0, out1)
    gs = (gs0, gs1)
    ss = (ss0, ss1)

    pltpu.sync_copy(z_ref, acc.at[pl.ds(sid * RPT, RPT)])
    pltpu.sync_copy(src_ref.at[pl.ds(wid * EPT, EPT)], src_v)
    pltpu.sync_copy(dst_ref.at[pl.ds(wid * EPT, EPT)], dst_v)
    plsc.subcore_barrier()

    def g_start(cc, b):
        off = cc * C
        pltpu.async_copy(t2_ref.at[src_v.at[pl.ds(off, C)]], rows[b], gs[b])
        pltpu.async_copy(a2_ref.at[dst_v.at[pl.ds(off, C)]], ad[b], gs[b])

    def g_wait(b):
        pltpu.make_async_copy(t2_ref.at[pl.ds(0, C)], rows[b], gs[b]).wait()
        pltpu.make_async_copy(a2_ref.at[pl.ds(0, C)], ad[b], gs[b]).wait()

    def s_start(b):
        pltpu.async_copy(out[b], acc.at[dstc[b]], ss[b], add=True)

    def s_wait(b):
        pltpu.make_async_copy(out[b], acc.at[dstc[b]], ss[b]).wait()

    def compute(cc, b):
        lanes = _iota16()
        c40 = jnp.full((16,), 40, jnp.int32)
        c0 = jnp.zeros((16,), jnp.int32)
        off = cc * C
        for j in range(C // 16):
            dstc[b][pl.ds(j * 16, 16)] = dst_v[pl.ds(off + j * 16, 16)]
        for g in range(C // 16):
            idx_r = g * 16 + lanes
            als = plsc.load_gather(rows[b], [idx_r, c40])
            ald = plsc.load_gather(ad[b], [idx_r, c0])
            t = als + ald
            w = jnp.exp(jnp.maximum(t, 0.2 * t))
            for e in range(16):
                row = g * 16 + e
                wb = _perm16(w, jnp.full((16,), e, jnp.int32))
                for kk in range(3):
                    hv = rows[b][row, pl.ds(kk * 16, 16)]
                    out[b][row, pl.ds(kk * 16, 16)] = hv * wb
            plsc.store_scatter(out[b], [idx_r, c40], w)

    g_start(0, 0)
    g_start(1, 1)

    def step(k2, _):
        for b in range(2):
            cc = 2 * k2 + b

            @pl.when(cc < NCH)
            def _():
                g_wait(b)

                @pl.when(cc >= 2)
                def _():
                    s_wait(b)

                compute(cc, b)
                s_start(b)

                @pl.when(cc + 2 < NCH)
                def _():
                    g_start(cc + 2, b)

        return 0

    lax.fori_loop(0, (NCH + 1) // 2, step, 0)
    s_wait(1)
    s_wait(0)
    plsc.subcore_barrier()
    base = sid * RPT
    pltpu.sync_copy(acc.at[pl.ds(base, RPT)],
                    out_ref.at[cid, pl.ds(base, RPT)])


def kernel(x, edge_index, W1, a_src1, a_dst1, b1, W2, a_src2, a_dst2, b2):
    f32 = jnp.float32
    sel = (jnp.arange(64)[:, None] // 8) == jnp.arange(8)[None, :]
    As = jnp.where(sel, a_src1.reshape(64)[:, None], 0.0).astype(f32)
    Ad = jnp.where(sel, a_dst1.reshape(64)[:, None], 0.0).astype(f32)
    E8 = sel.astype(f32).T

    T1, A1 = pl.pallas_call(
        _tc1_body,
        grid=(N // _BN,),
        in_specs=[
            pl.BlockSpec((_BN, NFEAT), lambda i: (i, 0)),
            pl.BlockSpec((NFEAT, 64), lambda i: (0, 0)),
            pl.BlockSpec((64, 8), lambda i: (0, 0)),
            pl.BlockSpec((64, 8), lambda i: (0, 0)),
        ],
        out_specs=[
            pl.BlockSpec((_BN, ROWS1), lambda i: (i, 0)),
            pl.BlockSpec((_BN, 8), lambda i: (i, 0)),
        ],
        out_shape=[
            jax.ShapeDtypeStruct((N, ROWS1), f32),
            jax.ShapeDtypeStruct((N, 8), f32),
        ],
    )(x, W1, As, Ad)

    src = edge_index[0]
    dst = edge_index[1]
    z1 = jnp.zeros((RPT, ROWS1), f32)
    acc1 = _sc1(src, dst, T1, A1, z1)

    T2, A2 = pl.pallas_call(
        _tc2_body,
        grid=(N // _BN,),
        in_specs=[
            pl.BlockSpec((2, _BN, ROWS1), lambda i: (0, i, 0)),
            pl.BlockSpec((1, 64), lambda i: (0, 0)),
            pl.BlockSpec((64, NCLASS), lambda i: (0, 0)),
            pl.BlockSpec((1, NCLASS), lambda i: (0, 0)),
            pl.BlockSpec((1, NCLASS), lambda i: (0, 0)),
            pl.BlockSpec((8, 64), lambda i: (0, 0)),
        ],
        out_specs=[
            pl.BlockSpec((_BN, ROWS2), lambda i: (i, 0)),
            pl.BlockSpec((_BN, 8), lambda i: (i, 0)),
        ],
        out_shape=[
            jax.ShapeDtypeStruct((N, ROWS2), f32),
            jax.ShapeDtypeStruct((N, 8), f32),
        ],
    )(acc1, b1.reshape(1, 64), W2, a_src2, a_dst2, E8)

    z2 = jnp.zeros((RPT, ROWS2), f32)
    acc2 = _sc2(src, dst, T2, A2, z2)

    out = pl.pallas_call(
        _tc3_body,
        grid=(N // _BN,),
        in_specs=[
            pl.BlockSpec((2, _BN, ROWS2), lambda i: (0, i, 0)),
            pl.BlockSpec((1, NCLASS), lambda i: (0, 0)),
        ],
        out_specs=pl.BlockSpec((_BN, NCLASS), lambda i: (i, 0)),
        out_shape=jax.ShapeDtypeStruct((N, NCLASS), f32),
    )(acc2, b2.reshape(1, NCLASS))

    return out


# speedup vs baseline: 157.1711x; 1.3006x over previous
import functools

import jax
import jax.numpy as jnp
from jax import lax
from jax.experimental import pallas as pl
from jax.experimental.pallas import tpu as pltpu
from jax.experimental.pallas import tpu_sc as plsc

N = 10000
E = 320000
NFEAT = 128
NHID = 8
NHEADS = 8
NCLASS = 40

NTILE = 32
EPT = E // NTILE
C = 80
NCH = EPT // C
ROWS1 = 72
ROWS2 = 48
NP = 10240
RPT = NP // 16

_BN = 1000


def _tc1_body(x_ref, w1_ref, as_ref, ad_ref, t1_ref, a1_ref):
    h = jnp.dot(x_ref[...], w1_ref[...], preferred_element_type=jnp.float32)
    als = jnp.dot(h, as_ref[...], preferred_element_type=jnp.float32)
    ald = jnp.dot(h, ad_ref[...], preferred_element_type=jnp.float32)
    t1_ref[...] = jnp.concatenate([h, als], axis=1)
    a1_ref[...] = ald


def _tc2_body(acc_ref, b1_ref, w2_ref, as2_ref, ad2_ref, e8_ref, t2_ref, a2_ref):
    a0 = acc_ref[0]
    a1 = acc_ref[1]
    num = a0[:, :64] + a1[:, :64]
    den = a0[:, 64:] + a1[:, 64:]
    r = 1.0 / (den + 1e-16)
    rexp = jnp.dot(r, e8_ref[...], preferred_element_type=jnp.float32)
    hid = num * rexp + b1_ref[...]
    hid = jnp.where(hid > 0, hid, jnp.exp(hid) - 1.0)
    h2 = jnp.dot(hid, w2_ref[...], preferred_element_type=jnp.float32)
    als2 = jnp.sum(h2 * as2_ref[...], axis=1, keepdims=True)
    ald2 = jnp.sum(h2 * ad2_ref[...], axis=1, keepdims=True)
    pad = jnp.zeros((h2.shape[0], 7), jnp.float32)
    t2_ref[...] = jnp.concatenate([h2, als2, pad], axis=1)
    a2_ref[...] = jnp.broadcast_to(ald2, (h2.shape[0], 8))


def _tc3_body(acc_ref, b2_ref, out_ref):
    a0 = acc_ref[0]
    a1 = acc_ref[1]
    num = a0[:, :40] + a1[:, :40]
    den = a0[:, 40:41] + a1[:, 40:41]
    o = num / (den + 1e-16) + b2_ref[...]
    m = jnp.max(o, axis=1, keepdims=True)
    s = jnp.sum(jnp.exp(o - m), axis=1, keepdims=True)
    out_ref[...] = o - m - jnp.log(s)


def _iota16():
    return lax.broadcasted_iota(jnp.int32, (16,), 0)


def _perm16(x, idx):
    return lax.gather(
        x, idx[:, None],
        lax.GatherDimensionNumbers(offset_dims=(), collapsed_slice_dims=(0,),
                                   start_index_map=(0,)),
        (1,), mode=lax.GatherScatterMode.PROMISE_IN_BOUNDS)


_mesh = plsc.VectorSubcoreMesh(core_axis_name="c", subcore_axis_name="s",
                               num_cores=2, num_subcores=16)


@functools.partial(
    pl.kernel,
    out_type=jax.ShapeDtypeStruct((2, NP, ROWS1), jnp.float32),
    mesh=_mesh,
    compiler_params=pltpu.CompilerParams(use_tc_tiling_on_sc=False,
                                         needs_layout_passes=False),
    scratch_types=[
        pltpu.VMEM_SHARED((NP, ROWS1), jnp.float32),
        pltpu.VMEM((EPT,), jnp.int32),
        pltpu.VMEM((EPT,), jnp.int32),
        pltpu.VMEM((C,), jnp.int32),
        pltpu.VMEM((C,), jnp.int32),
        pltpu.VMEM((C, ROWS1), jnp.float32),
        pltpu.VMEM((C, ROWS1), jnp.float32),
        pltpu.VMEM((C, 8), jnp.float32),
        pltpu.VMEM((C, 8), jnp.float32),
        pltpu.VMEM((C, ROWS1), jnp.float32),
        pltpu.VMEM((C, ROWS1), jnp.float32),
        pltpu.VMEM((C, 16), jnp.float32),
        pltpu.SemaphoreType.DMA,
        pltpu.SemaphoreType.DMA,
        pltpu.SemaphoreType.DMA,
        pltpu.SemaphoreType.DMA,
    ],
)
def _sc1(src_ref, dst_ref, t1_ref, a1_ref, z_ref, out_ref,
         acc, src_v, dst_v, dstc0, dstc1, rows0, rows1, ad0, ad1,
         out0, out1, w2_v, gs0, gs1, ss0, ss1):
    cid = lax.axis_index("c")
    sid = lax.axis_index("s")
    wid = cid * 16 + sid
    dstc = (dstc0, dstc1)
    rows = (rows0, rows1)
    ad = (ad0, ad1)
    out = (out0, out1)
    gs = (gs0, gs1)
    ss = (ss0, ss1)

    pltpu.sync_copy(z_ref, acc.at[pl.ds(sid * RPT, RPT)])

    pltpu.sync_copy(src_ref.at[pl.ds(wid * EPT, EPT)], src_v)
    pltpu.sync_copy(dst_ref.at[pl.ds(wid * EPT, EPT)], dst_v)
    plsc.subcore_barrier()

    def g_start(cc, b):
        off = cc * C
        pltpu.async_copy(t1_ref.at[src_v.at[pl.ds(off, C)]], rows[b], gs[b])
        pltpu.async_copy(a1_ref.at[dst_v.at[pl.ds(off, C)]], ad[b], gs[b])

    def g_wait(b):
        pltpu.make_async_copy(t1_ref.at[pl.ds(0, C)], rows[b], gs[b]).wait()
        pltpu.make_async_copy(a1_ref.at[pl.ds(0, C)], ad[b], gs[b]).wait()

    def s_start(b):
        pltpu.async_copy(out[b], acc.at[dstc[b]], ss[b], add=True)

    def s_wait(b):
        pltpu.make_async_copy(out[b], acc.at[dstc[b]], ss[b]).wait()

    def compute(cc, b):
        lanes = _iota16()
        off = cc * C
        for j in range(C // 16):
            dstc[b][pl.ds(j * 16, 16)] = dst_v[pl.ds(off + j * 16, 16)]
        for g in range(C // 16):
            er = g * 16 + lanes
            als = [plsc.load_gather(rows[b], [er, jnp.full((16,), 64 + hh, jnp.int32)])
                   for hh in range(8)]
            ald = [plsc.load_gather(ad[b], [er, jnp.full((16,), hh, jnp.int32)])
                   for hh in range(8)]
            ws = []
            for hh in range(8):
                t = als[hh] + ald[hh]
                ws.append(jnp.exp(jnp.maximum(t, 0.2 * t)))
            for hh in range(8):
                ch = jnp.full((16,), hh, jnp.int32)
                plsc.store_scatter(out[b], [er, 64 + ch], ws[hh])
                plsc.store_scatter(w2_v, [er, ch], ws[hh])
                plsc.store_scatter(w2_v, [er, 8 + ch], ws[hh])
        for e in range(C):
            wb = w2_v[e, pl.ds(0, 16)]
            for kk in range(4):
                hv = rows[b][e, pl.ds(kk * 16, 16)]
                out[b][e, pl.ds(kk * 16, 16)] = hv * wb

    g_start(0, 0)
    g_start(1, 1)

    def step(k2, _):
        for b in range(2):
            cc = 2 * k2 + b

            @pl.when(cc < NCH)
            def _():
                g_wait(b)

                @pl.when(cc >= 2)
                def _():
                    s_wait(b)

                compute(cc, b)
                s_start(b)

                @pl.when(cc + 2 < NCH)
                def _():
                    g_start(cc + 2, b)

        return 0

    lax.fori_loop(0, (NCH + 1) // 2, step, 0)
    s_wait(1)
    s_wait(0)
    plsc.subcore_barrier()
    base = sid * RPT
    pltpu.sync_copy(acc.at[pl.ds(base, RPT)],
                    out_ref.at[cid, pl.ds(base, RPT)])


@functools.partial(
    pl.kernel,
    out_type=jax.ShapeDtypeStruct((2, NP, ROWS2), jnp.float32),
    mesh=_mesh,
    compiler_params=pltpu.CompilerParams(use_tc_tiling_on_sc=False,
                                         needs_layout_passes=False),
    scratch_types=[
        pltpu.VMEM_SHARED((NP, ROWS2), jnp.float32),
        pltpu.VMEM((EPT,), jnp.int32),
        pltpu.VMEM((EPT,), jnp.int32),
        pltpu.VMEM((C,), jnp.int32),
        pltpu.VMEM((C,), jnp.int32),
        pltpu.VMEM((C, ROWS2), jnp.float32),
        pltpu.VMEM((C, ROWS2), jnp.float32),
        pltpu.VMEM((C, 8), jnp.float32),
        pltpu.VMEM((C, 8), jnp.float32),
        pltpu.VMEM((C, ROWS2), jnp.float32),
        pltpu.VMEM((C, ROWS2), jnp.float32),
        pltpu.SemaphoreType.DMA,
        pltpu.SemaphoreType.DMA,
        pltpu.SemaphoreType.DMA,
        pltpu.SemaphoreType.DMA,
    ],
)
def _sc2(src_ref, dst_ref, t2_ref, a2_ref, z_ref, out_ref,
         acc, src_v, dst_v, dstc0, dstc1, rows0, rows1, ad0, ad1,
         out0, out1, gs0, gs1, ss0, ss1):
    cid = lax.axis_index("c")
    sid = lax.axis_index("s")
    wid = cid * 16 + sid
    dstc = (dstc0, dstc1)
    rows = (rows0, rows1)
    ad = (ad0, ad1)
    out = (out0, out1)
    gs = (gs0, gs1)
    ss = (ss0, ss1)

    pltpu.sync_copy(z_ref, acc.at[pl.ds(sid * RPT, RPT)])
    pltpu.sync_copy(src_ref.at[pl.ds(wid * EPT, EPT)], src_v)
    pltpu.sync_copy(dst_ref.at[pl.ds(wid * EPT, EPT)], dst_v)
    plsc.subcore_barrier()

    def g_start(cc, b):
        off = cc * C
        pltpu.async_copy(t2_ref.at[src_v.at[pl.ds(off, C)]], rows[b], gs[b])
        pltpu.async_copy(a2_ref.at[dst_v.at[pl.ds(off, C)]], ad[b], gs[b])

    def g_wait(b):
        pltpu.make_async_copy(t2_ref.at[pl.ds(0, C)], rows[b], gs[b]).wait()
        pltpu.make_async_copy(a2_ref.at[pl.ds(0, C)], ad[b], gs[b]).wait()

    def s_start(b):
        pltpu.async_copy(out[b], acc.at[dstc[b]], ss[b], add=True)

    def s_wait(b):
        pltpu.make_async_copy(out[b], acc.at[dstc[b]], ss[b]).wait()

    def compute(cc, b):
        lanes = _iota16()
        c40 = jnp.full((16,), 40, jnp.int32)
        off = cc * C
        for j in range(C // 16):
            dstc[b][pl.ds(j * 16, 16)] = dst_v[pl.ds(off + j * 16, 16)]
        ers = [g * 16 + lanes for g in range(C // 16)]
        als = [plsc.load_gather(rows[b], [er, c40]) for er in ers]
        ald = [plsc.load_gather(ad[b], [er, c40 - 40]) for er in ers]
        ws = []
        for g in range(C // 16):
            t = als[g] + ald[g]
            ws.append(jnp.exp(jnp.maximum(t, 0.2 * t)))
        for e in range(C):
            wb = _perm16(ws[e // 16], jnp.full((16,), e % 16, jnp.int32))
            for kk in range(3):
                hv = rows[b][e, pl.ds(kk * 16, 16)]
                out[b][e, pl.ds(kk * 16, 16)] = hv * wb
        for g in range(C // 16):
            plsc.store_scatter(out[b], [ers[g], c40], ws[g])

    g_start(0, 0)
    g_start(1, 1)

    def step(k2, _):
        for b in range(2):
            cc = 2 * k2 + b

            @pl.when(cc < NCH)
            def _():
                g_wait(b)

                @pl.when(cc >= 2)
                def _():
                    s_wait(b)

                compute(cc, b)
                s_start(b)

                @pl.when(cc + 2 < NCH)
                def _():
                    g_start(cc + 2, b)

        return 0

    lax.fori_loop(0, (NCH + 1) // 2, step, 0)
    s_wait(1)
    s_wait(0)
    plsc.subcore_barrier()
    base = sid * RPT
    pltpu.sync_copy(acc.at[pl.ds(base, RPT)],
                    out_ref.at[cid, pl.ds(base, RPT)])


def kernel(x, edge_index, W1, a_src1, a_dst1, b1, W2, a_src2, a_dst2, b2):
    f32 = jnp.float32
    c = jnp.arange(64)
    P64 = (c[:, None] == ((c % 8) * 8 + c // 8)[None, :]).astype(f32)
    W1t = W1 @ P64
    selt = (jnp.arange(64)[:, None] % 8) == jnp.arange(8)[None, :]
    As = jnp.where(selt, jnp.take(a_src1.reshape(64), (c % 8) * 8 + c // 8)[:, None], 0.0).astype(f32)
    Ad = jnp.where(selt, jnp.take(a_dst1.reshape(64), (c % 8) * 8 + c // 8)[:, None], 0.0).astype(f32)
    E8 = selt.astype(f32).T
    b1t = jnp.take(b1, (c % 8) * 8 + c // 8)
    W2e = P64.T @ W2

    T1, A1 = pl.pallas_call(
        _tc1_body,
        grid=(N // _BN,),
        in_specs=[
            pl.BlockSpec((_BN, NFEAT), lambda i: (i, 0)),
            pl.BlockSpec((NFEAT, 64), lambda i: (0, 0)),
            pl.BlockSpec((64, 8), lambda i: (0, 0)),
            pl.BlockSpec((64, 8), lambda i: (0, 0)),
        ],
        out_specs=[
            pl.BlockSpec((_BN, ROWS1), lambda i: (i, 0)),
            pl.BlockSpec((_BN, 8), lambda i: (i, 0)),
        ],
        out_shape=[
            jax.ShapeDtypeStruct((N, ROWS1), f32),
            jax.ShapeDtypeStruct((N, 8), f32),
        ],
    )(x, W1t, As, Ad)

    src = edge_index[0]
    dst = edge_index[1]
    z1 = jnp.zeros((RPT, ROWS1), f32)
    acc1 = _sc1(src, dst, T1, A1, z1)

    T2, A2 = pl.pallas_call(
        _tc2_body,
        grid=(N // _BN,),
        in_specs=[
            pl.BlockSpec((2, _BN, ROWS1), lambda i: (0, i, 0)),
            pl.BlockSpec((1, 64), lambda i: (0, 0)),
            pl.BlockSpec((64, NCLASS), lambda i: (0, 0)),
            pl.BlockSpec((1, NCLASS), lambda i: (0, 0)),
            pl.BlockSpec((1, NCLASS), lambda i: (0, 0)),
            pl.BlockSpec((8, 64), lambda i: (0, 0)),
        ],
        out_specs=[
            pl.BlockSpec((_BN, ROWS2), lambda i: (i, 0)),
            pl.BlockSpec((_BN, 8), lambda i: (i, 0)),
        ],
        out_shape=[
            jax.ShapeDtypeStruct((N, ROWS2), f32),
            jax.ShapeDtypeStruct((N, 8), f32),
        ],
    )(acc1, b1t.reshape(1, 64), W2e, a_src2, a_dst2, E8)

    z2 = jnp.zeros((RPT, ROWS2), f32)
    acc2 = _sc2(src, dst, T2, A2, z2)

    out = pl.pallas_call(
        _tc3_body,
        grid=(N // _BN,),
        in_specs=[
            pl.BlockSpec((2, _BN, ROWS2), lambda i: (0, i, 0)),
            pl.BlockSpec((1, NCLASS), lambda i: (0, 0)),
        ],
        out_specs=pl.BlockSpec((_BN, NCLASS), lambda i: (i, 0)),
        out_shape=jax.ShapeDtypeStruct((N, NCLASS), f32),
    )(acc2, b2.reshape(1, NCLASS))

    return out
